# baseline (speedup 1.0000x reference)
import functools
import math

import jax
import jax.numpy as jnp
from jax import lax
from jax.experimental import pallas as pl
from jax.experimental.pallas import tpu as pltpu
from jax.experimental.pallas import tpu_sc as plsc

D_MODEL = 2048
SEQ = 2048
BATCH = 4
NW = 32
PW = SEQ // NW
C = 16
PC = PW // C
LANES = 16
SCALE = math.sqrt(float(D_MODEL))


def _sc_embed_body(x_ref, pos_ref, table_ref, out_ref, idx_v, pos_v, rows_v, sem):
    cid = lax.axis_index("c")
    sid = lax.axis_index("s")
    wid = sid * 2 + cid
    s0 = wid * PW

    for b in range(BATCH):
        pltpu.sync_copy(x_ref.at[pl.ds(b * SEQ + s0, PW)],
                        idx_v.at[pl.ds(b * PW, PW)])

    for pc in range(PC):
        pltpu.sync_copy(pos_ref.at[pl.ds(s0 + pc * C, C)], pos_v)
        for b in range(BATCH):
            pltpu.async_copy(
                table_ref.at[idx_v.at[pl.ds(b * PW + pc * C, C)]],
                rows_v, sem).wait()

            @plsc.parallel_loop(0, C * D_MODEL // LANES, unroll=8)
            def _compute(i):
                r = i >> 7
                off = pl.multiple_of((i & 127) << 4, LANES)
                rows_v[r, pl.ds(off, LANES)] = (
                    rows_v[r, pl.ds(off, LANES)] * SCALE
                    + pos_v[r, pl.ds(off, LANES)])

            pltpu.sync_copy(rows_v,
                            out_ref.at[pl.ds(b * SEQ + s0 + pc * C, C)])


@functools.cache
def _sc_embed():
    mesh = plsc.VectorSubcoreMesh(core_axis_name="c", subcore_axis_name="s")
    return pl.kernel(
        _sc_embed_body,
        out_type=jax.ShapeDtypeStruct((BATCH * SEQ, D_MODEL), jnp.float32),
        mesh=mesh,
        scratch_types=[
            pltpu.VMEM((BATCH * PW,), jnp.int32),
            pltpu.VMEM((C, D_MODEL), jnp.float32),
            pltpu.VMEM((C, D_MODEL), jnp.float32),
            pltpu.SemaphoreType.DMA,
        ],
    )


def kernel(x, table, pos_encoding, training=False):
    del training
    b, s = x.shape
    d = table.shape[1]
    xf = x.reshape(b * s)
    pos2d = pos_encoding[0, :s, :]
    out = _sc_embed()(xf, pos2d, table)
    return out.reshape(b, s, d)


# speedup vs baseline: 1.5215x; 1.5215x over previous
import functools
import math

import jax
import jax.numpy as jnp
from jax import lax
from jax.experimental import pallas as pl
from jax.experimental.pallas import tpu as pltpu
from jax.experimental.pallas import tpu_sc as plsc

D_MODEL = 2048
SEQ = 2048
BATCH = 4
NW = 32
PW = SEQ // NW
C = 8
PC = PW // C
CH = PC * BATCH
NB = 4
PF = 2
LANES = 16
SCALE = math.sqrt(float(D_MODEL))


def _sc_embed_body(x_ref, pos_ref, table_ref, out_ref,
                   idx_v, rows_v, pos_v, gsems, wsems, psems):
    cid = lax.axis_index("c")
    sid = lax.axis_index("s")
    wid = sid * 2 + cid
    s0 = wid * PW

    for b in range(BATCH):
        pltpu.sync_copy(x_ref.at[pl.ds(b * SEQ + s0, PW)],
                        idx_v.at[pl.ds(b * PW, PW)])

    def fire_pos(pc):
        return pltpu.async_copy(pos_ref.at[pl.ds(s0 + pc * C, C)],
                                pos_v.at[pc % 2], psems[pc % 2])

    def fire_gather(c):
        b, pc = c % BATCH, c // BATCH
        return pltpu.async_copy(
            table_ref.at[idx_v.at[pl.ds(b * PW + pc * C, C)]],
            rows_v.at[c % NB], gsems[c % NB])

    def fire_write(c):
        b, pc = c % BATCH, c // BATCH
        return pltpu.async_copy(rows_v.at[c % NB],
                                out_ref.at[pl.ds(b * SEQ + s0 + pc * C, C)],
                                wsems[c % NB])

    pos_d = {0: fire_pos(0)}
    gather_d, write_d = {}, {}
    for t in range(CH + PF):
        if t < CH:
            if t >= NB:
                write_d[t - NB].wait()
            gather_d[t] = fire_gather(t)
        c = t - PF
        if c < 0:
            continue
        gather_d[c].wait()
        b, pc = c % BATCH, c // BATCH
        if b == 0:
            pos_d[pc].wait()
            if pc + 1 < PC:
                pos_d[pc + 1] = fire_pos(pc + 1)
        rbuf = rows_v.at[c % NB]
        pbuf = pos_v.at[pc % 2]

        @plsc.parallel_loop(0, C * D_MODEL // LANES, unroll=8)
        def _compute(i):
            r = i >> 7
            off = pl.multiple_of((i & 127) << 4, LANES)
            rbuf[r, pl.ds(off, LANES)] = (
                rbuf[r, pl.ds(off, LANES)] * SCALE
                + pbuf[r, pl.ds(off, LANES)])

        write_d[c] = fire_write(c)

    for c in range(CH - NB, CH):
        write_d[c].wait()


@functools.cache
def _sc_embed():
    mesh = plsc.VectorSubcoreMesh(core_axis_name="c", subcore_axis_name="s")
    return pl.kernel(
        _sc_embed_body,
        out_type=jax.ShapeDtypeStruct((BATCH * SEQ, D_MODEL), jnp.float32),
        mesh=mesh,
        scratch_types=[
            pltpu.VMEM((BATCH * PW,), jnp.int32),
            pltpu.VMEM((NB, C, D_MODEL), jnp.float32),
            pltpu.VMEM((2, C, D_MODEL), jnp.float32),
            [pltpu.SemaphoreType.DMA] * NB,
            [pltpu.SemaphoreType.DMA] * NB,
            [pltpu.SemaphoreType.DMA] * 2,
        ],
    )


def kernel(x, table, pos_encoding, training=False):
    del training
    b, s = x.shape
    d = table.shape[1]
    xf = x.reshape(b * s)
    pos2d = pos_encoding[0, :s, :]
    out = _sc_embed()(xf, pos2d, table)
    return out.reshape(b, s, d)


# speedup vs baseline: 1.6871x; 1.1088x over previous
import functools
import math

import jax
import jax.numpy as jnp
from jax import lax
from jax.experimental import pallas as pl
from jax.experimental.pallas import tpu as pltpu
from jax.experimental.pallas import tpu_sc as plsc

D_MODEL = 2048
SEQ = 2048
BATCH = 4
NW = 32
PW = SEQ // NW
C = 8
PC = PW // C
CH = PC * BATCH
NB = 5
PF = 3
LANES = 16
SCALE = math.sqrt(float(D_MODEL))


def _sc_embed_body(x_ref, pos_ref, table_ref, out_ref,
                   idx_v, rows_v, pos_v, gsems, wsems, psems):
    cid = lax.axis_index("c")
    sid = lax.axis_index("s")
    wid = sid * 2 + cid
    s0 = wid * PW

    for b in range(BATCH):
        pltpu.sync_copy(x_ref.at[pl.ds(b * SEQ + s0, PW)],
                        idx_v.at[pl.ds(b * PW, PW)])

    def fire_pos(pc):
        return pltpu.async_copy(pos_ref.at[pl.ds(s0 + pc * C, C)],
                                pos_v.at[pc % 2], psems[pc % 2])

    def fire_gather(c):
        b, pc = c % BATCH, c // BATCH
        return pltpu.async_copy(
            table_ref.at[idx_v.at[pl.ds(b * PW + pc * C, C)]],
            rows_v.at[c % NB], gsems[c % NB])

    def fire_write(c):
        b, pc = c % BATCH, c // BATCH
        return pltpu.async_copy(rows_v.at[c % NB],
                                out_ref.at[pl.ds(b * SEQ + s0 + pc * C, C)],
                                wsems[c % NB])

    pos_d = {0: fire_pos(0)}
    gather_d, write_d = {}, {}
    for t in range(CH + PF):
        if t < CH:
            if t >= NB:
                write_d[t - NB].wait()
            gather_d[t] = fire_gather(t)
        c = t - PF
        if c < 0:
            continue
        gather_d[c].wait()
        b, pc = c % BATCH, c // BATCH
        if b == 0:
            pos_d[pc].wait()
            if pc + 1 < PC:
                pos_d[pc + 1] = fire_pos(pc + 1)
        rbuf = rows_v.at[c % NB]
        pbuf = pos_v.at[pc % 2]

        @plsc.parallel_loop(0, C * D_MODEL // LANES, unroll=8)
        def _compute(i):
            r = i >> 7
            off = pl.multiple_of((i & 127) << 4, LANES)
            rbuf[r, pl.ds(off, LANES)] = (
                rbuf[r, pl.ds(off, LANES)] * SCALE
                + pbuf[r, pl.ds(off, LANES)])

        write_d[c] = fire_write(c)

    for c in range(CH - NB, CH):
        write_d[c].wait()


@functools.cache
def _sc_embed():
    mesh = plsc.VectorSubcoreMesh(core_axis_name="c", subcore_axis_name="s")
    return pl.kernel(
        _sc_embed_body,
        out_type=jax.ShapeDtypeStruct((BATCH * SEQ, D_MODEL), jnp.float32),
        mesh=mesh,
        scratch_types=[
            pltpu.VMEM((BATCH * PW,), jnp.int32),
            pltpu.VMEM((NB, C, D_MODEL), jnp.float32),
            pltpu.VMEM((2, C, D_MODEL), jnp.float32),
            [pltpu.SemaphoreType.DMA] * NB,
            [pltpu.SemaphoreType.DMA] * NB,
            [pltpu.SemaphoreType.DMA] * 2,
        ],
    )


def kernel(x, table, pos_encoding, training=False):
    del training
    b, s = x.shape
    d = table.shape[1]
    xf = x.reshape(b * s)
    pos2d = pos_encoding.reshape(pos_encoding.shape[1], d)
    out = _sc_embed()(xf, pos2d, table)
    return out.reshape(b, s, d)


# speedup vs baseline: 1.7014x; 1.0085x over previous
import functools
import math

import jax
import jax.numpy as jnp
from jax import lax
from jax.experimental import pallas as pl
from jax.experimental.pallas import tpu as pltpu
from jax.experimental.pallas import tpu_sc as plsc

D_MODEL = 2048
SEQ = 2048
BATCH = 4
NW = 32
PW = SEQ // NW
C = 8
PC = PW // C
CH = PC * BATCH
NB = 5
PF = 3
LANES = 16
SCALE = math.sqrt(float(D_MODEL))


def _sc_embed_body(x_ref, pos_ref, table_ref, out_ref,
                   idx_v, rows_v, pos_v, gsems, wsems, psems):
    cid = lax.axis_index("c")
    sid = lax.axis_index("s")
    wid = sid * 2 + cid
    s0 = wid * PW

    for b in range(BATCH):
        pltpu.sync_copy(x_ref.at[b, pl.ds(s0, PW)],
                        idx_v.at[pl.ds(b * PW, PW)])

    def fire_pos(pc):
        return pltpu.async_copy(pos_ref.at[0, pl.ds(s0 + pc * C, C), :],
                                pos_v.at[pc % 2], psems[pc % 2])

    def fire_gather(c):
        b, pc = c % BATCH, c // BATCH
        return pltpu.async_copy(
            table_ref.at[idx_v.at[pl.ds(b * PW + pc * C, C)]],
            rows_v.at[c % NB], gsems[c % NB])

    def fire_write(c):
        b, pc = c % BATCH, c // BATCH
        return pltpu.async_copy(rows_v.at[c % NB],
                                out_ref.at[b, pl.ds(s0 + pc * C, C), :],
                                wsems[c % NB])

    pos_d = {0: fire_pos(0)}
    gather_d, write_d = {}, {}
    for t in range(CH + PF):
        if t < CH:
            if t >= NB:
                write_d[t - NB].wait()
            gather_d[t] = fire_gather(t)
        c = t - PF
        if c < 0:
            continue
        gather_d[c].wait()
        b, pc = c % BATCH, c // BATCH
        if b == 0:
            pos_d[pc].wait()
            if pc + 1 < PC:
                pos_d[pc + 1] = fire_pos(pc + 1)
        rbuf = rows_v.at[c % NB]
        pbuf = pos_v.at[pc % 2]

        @plsc.parallel_loop(0, C * D_MODEL // LANES, unroll=8)
        def _compute(i):
            r = i >> 7
            off = pl.multiple_of((i & 127) << 4, LANES)
            rbuf[r, pl.ds(off, LANES)] = (
                rbuf[r, pl.ds(off, LANES)] * SCALE
                + pbuf[r, pl.ds(off, LANES)])

        write_d[c] = fire_write(c)

    for c in range(CH - NB, CH):
        write_d[c].wait()


@functools.cache
def _sc_embed():
    mesh = plsc.VectorSubcoreMesh(core_axis_name="c", subcore_axis_name="s")
    return pl.kernel(
        _sc_embed_body,
        out_type=jax.ShapeDtypeStruct((BATCH, SEQ, D_MODEL), jnp.float32),
        mesh=mesh,
        scratch_types=[
            pltpu.VMEM((BATCH * PW,), jnp.int32),
            pltpu.VMEM((NB, C, D_MODEL), jnp.float32),
            pltpu.VMEM((2, C, D_MODEL), jnp.float32),
            [pltpu.SemaphoreType.DMA] * NB,
            [pltpu.SemaphoreType.DMA] * NB,
            [pltpu.SemaphoreType.DMA] * 2,
        ],
    )


def kernel(x, table, pos_encoding, training=False):
    del training
    return _sc_embed()(x, pos_encoding, table)


# speedup vs baseline: 1.7196x; 1.0107x over previous
import functools
import math

import jax
import jax.numpy as jnp
from jax import lax
from jax.experimental import pallas as pl
from jax.experimental.pallas import tpu as pltpu
from jax.experimental.pallas import tpu_sc as plsc

D_MODEL = 2048
SEQ = 2048
BATCH = 4
NW = 32
PW = SEQ // NW
C = 8
PC = PW // C
CH = PC * BATCH
NB = 5
PF = 3
LANES = 16
SCALE = math.sqrt(float(D_MODEL))


def _sc_embed_body(x_ref, pos_ref, table_ref, out_ref,
                   idx_v, rows_v, pos_v, gsems, wsems, psems):
    cid = lax.axis_index("c")
    sid = lax.axis_index("s")
    wid = sid * 2 + cid
    s0 = wid * PW

    def fire_pos(pc):
        return pltpu.async_copy(pos_ref.at[0, pl.ds(s0 + pc * C, C), :],
                                pos_v.at[pc % 2], psems[pc % 2])

    def fire_gather(c):
        b, pc = c % BATCH, c // BATCH
        return pltpu.async_copy(
            table_ref.at[idx_v.at[pl.ds(b * PW + pc * C, C)]],
            rows_v.at[c % NB], gsems[c % NB])

    def fire_write(c):
        b, pc = c % BATCH, c // BATCH
        return pltpu.async_copy(rows_v.at[c % NB],
                                out_ref.at[b, pl.ds(s0 + pc * C, C), :],
                                wsems[c % NB])

    pos_d = {0: fire_pos(0)}

    idx_d = [pltpu.async_copy(x_ref.at[b, pl.ds(s0, PW)],
                              idx_v.at[pl.ds(b * PW, PW)], wsems[b])
             for b in range(BATCH)]
    for d in idx_d:
        d.wait()

    gather_d, write_d = {}, {}
    for t in range(CH + PF):
        if t < CH:
            if t >= NB:
                write_d[t - NB].wait()
            gather_d[t] = fire_gather(t)
        c = t - PF
        if c < 0:
            continue
        gather_d[c].wait()
        b, pc = c % BATCH, c // BATCH
        if b == 0:
            pos_d[pc].wait()
            if pc + 1 < PC:
                pos_d[pc + 1] = fire_pos(pc + 1)
        rbuf = rows_v.at[c % NB]
        pbuf = pos_v.at[pc % 2]

        @plsc.parallel_loop(0, C * D_MODEL // LANES, unroll=8)
        def _compute(i):
            r = i >> 7
            off = pl.multiple_of((i & 127) << 4, LANES)
            rbuf[r, pl.ds(off, LANES)] = (
                rbuf[r, pl.ds(off, LANES)] * SCALE
                + pbuf[r, pl.ds(off, LANES)])

        write_d[c] = fire_write(c)

    for c in range(CH - NB, CH):
        write_d[c].wait()


@functools.cache
def _sc_embed():
    mesh = plsc.VectorSubcoreMesh(core_axis_name="c", subcore_axis_name="s")
    return pl.kernel(
        _sc_embed_body,
        out_type=jax.ShapeDtypeStruct((BATCH, SEQ, D_MODEL), jnp.float32),
        mesh=mesh,
        scratch_types=[
            pltpu.VMEM((BATCH * PW,), jnp.int32),
            pltpu.VMEM((NB, C, D_MODEL), jnp.float32),
            pltpu.VMEM((2, C, D_MODEL), jnp.float32),
            [pltpu.SemaphoreType.DMA] * NB,
            [pltpu.SemaphoreType.DMA] * NB,
            [pltpu.SemaphoreType.DMA] * 2,
        ],
    )


def kernel(x, table, pos_encoding, training=False):
    del training
    return _sc_embed()(x, pos_encoding, table)
